# Initial kernel scaffold; baseline (speedup 1.0000x reference)
import jax
import jax.numpy as jnp
from jax.experimental import pallas as pl


def kernel(feature, edge_index, emb_tables, fc_W, fc_b):
    raise NotImplementedError("write your pallas kernel here")


# speedup vs baseline: 5.8023x; 5.8023x over previous
import functools

import jax
import jax.numpy as jnp
from jax import lax
from jax.experimental import pallas as pl
from jax.experimental.pallas import tpu as pltpu
from jax.experimental.pallas import tpu_sc as plsc

NF = 19
NI = 128
NO = 256
VOCAB = 1000

NC = 2
NS = 16
NW = NC * NS
LANES = 16

CH = 4
ROWS = CH * NF
ROWSP = 80


def _proj_body(emb_ref, w_ref, b_ref, p_ref):
    f = pl.program_id(0)
    acc = jnp.dot(emb_ref[0], w_ref[0], preferred_element_type=jnp.float32)
    bias = jnp.where(f == 0, b_ref[0], jnp.zeros_like(b_ref[0]))
    p_ref[0] = acc + bias[None, :]


def _project_tables(emb_tables, fc_W, fc_b):
    w3 = fc_W.reshape(NF, NI, NO)
    b2 = fc_b.reshape(1, NO)
    p = pl.pallas_call(
        _proj_body,
        grid=(NF,),
        in_specs=[
            pl.BlockSpec((1, VOCAB, NI), lambda f: (f, 0, 0)),
            pl.BlockSpec((1, NI, NO), lambda f: (f, 0, 0)),
            pl.BlockSpec((1, NO), lambda f: (0, 0)),
        ],
        out_specs=pl.BlockSpec((1, VOCAB, NO), lambda f: (f, 0, 0)),
        out_shape=jax.ShapeDtypeStruct((NF, VOCAB, NO), jnp.float32),
    )(emb_tables, w3, b2)
    return p.reshape(NF * VOCAB, NO)


def _make_bag_kernel(bs):
    spw = bs // NW
    nchunk = spw // CH
    mesh = plsc.VectorSubcoreMesh(core_axis_name="c", subcore_axis_name="s")

    @functools.partial(
        pl.kernel,
        out_type=jax.ShapeDtypeStruct((bs, NO), jnp.float32),
        mesh=mesh,
        scratch_types=[
            pltpu.VMEM((nchunk, ROWSP), jnp.int32),
            pltpu.VMEM((ROWSP, NO), jnp.float32),
            pltpu.VMEM((ROWSP, NO), jnp.float32),
            pltpu.VMEM((spw, NO), jnp.float32),
            pltpu.SemaphoreType.DMA,
            pltpu.SemaphoreType.DMA,
        ],
    )
    def bag(idx_hbm, p_hbm, out_hbm, idx_v, rows0, rows1, out_v, sem0, sem1):
        cid = lax.axis_index("c")
        sid = lax.axis_index("s")
        w = sid * NC + cid

        pltpu.sync_copy(idx_hbm.at[w], idx_v)

        def compute_chunk(j, rows):
            def cc_body(cc, carry):
                col = pl.ds(cc * LANES, LANES)
                for s in range(CH):
                    acc = rows[s * NF, col]
                    for f in range(1, NF):
                        acc = acc + rows[s * NF + f, col]
                    out_v[j * CH + s, col] = acc
                return carry

            lax.fori_loop(0, NO // LANES, cc_body, 0)

        pltpu.async_copy(p_hbm.at[idx_v.at[0]], rows0, sem0)

        def loop_body(jj, carry):
            j0 = 2 * jj
            j1 = j0 + 1
            pltpu.async_copy(p_hbm.at[idx_v.at[j1]], rows1, sem1)
            pltpu.make_async_copy(p_hbm.at[idx_v.at[j0]], rows0, sem0).wait()
            compute_chunk(j0, rows0)

            @pl.when(jj < nchunk // 2 - 1)
            def _():
                pltpu.async_copy(p_hbm.at[idx_v.at[j0 + 2]], rows0, sem0)

            pltpu.make_async_copy(p_hbm.at[idx_v.at[j1]], rows1, sem1).wait()
            compute_chunk(j1, rows1)
            return carry

        lax.fori_loop(0, nchunk // 2, loop_body, 0)

        pltpu.sync_copy(out_v, out_hbm.at[pl.ds(w * spw, spw)])

    return bag


def kernel(feature, edge_index, emb_tables, fc_W, fc_b):
    del edge_index
    bs = feature.shape[0]
    spw = bs // NW
    nchunk = spw // CH

    flat = feature.astype(jnp.int32) + (jnp.arange(NF, dtype=jnp.int32) * VOCAB)[None, :]
    idx = flat.reshape(NW, nchunk, ROWS)
    idx = jnp.pad(idx, ((0, 0), (0, 0), (0, ROWSP - ROWS)))

    p = _project_tables(emb_tables, fc_W, fc_b)
    return _make_bag_kernel(bs)(idx, p)


# speedup vs baseline: 6.2047x; 1.0694x over previous
import functools

import jax
import jax.numpy as jnp
from jax import lax
from jax.experimental import pallas as pl
from jax.experimental.pallas import tpu as pltpu
from jax.experimental.pallas import tpu_sc as plsc

NF = 19
NI = 128
NO = 256
VOCAB = 1000

NC = 2
NS = 16
NW = NC * NS
LANES = 16

CH = 4
ROWS = CH * NF
ROWSP = 80


def _round_to_bf16_hi(bits):
    return bits + jnp.uint32(0x8000)


def _proj_body(emb_ref, we_ref, wo_ref, be_ref, bo_ref, p_ref):
    f = pl.program_id(0)
    acc_e = jnp.dot(emb_ref[0], we_ref[0], preferred_element_type=jnp.float32)
    acc_o = jnp.dot(emb_ref[0], wo_ref[0], preferred_element_type=jnp.float32)
    is0 = (f == 0).astype(jnp.float32)
    acc_e = acc_e + is0 * be_ref[0][None, :]
    acc_o = acc_o + is0 * bo_ref[0][None, :]
    be = _round_to_bf16_hi(lax.bitcast_convert_type(acc_e, jnp.uint32))
    bo = _round_to_bf16_hi(lax.bitcast_convert_type(acc_o, jnp.uint32))
    p_ref[0] = (be >> 16) | (bo & jnp.uint32(0xFFFF0000))


def _project_tables(emb_tables, fc_W, fc_b):
    w3 = fc_W.reshape(NF, NI, NO)
    w_e = w3[:, :, :NO // 2]
    w_o = w3[:, :, NO // 2:]
    b_e = fc_b[:NO // 2].reshape(1, NO // 2)
    b_o = fc_b[NO // 2:].reshape(1, NO // 2)
    p = pl.pallas_call(
        _proj_body,
        grid=(NF,),
        in_specs=[
            pl.BlockSpec((1, VOCAB, NI), lambda f: (f, 0, 0)),
            pl.BlockSpec((1, NI, NO // 2), lambda f: (f, 0, 0)),
            pl.BlockSpec((1, NI, NO // 2), lambda f: (f, 0, 0)),
            pl.BlockSpec((1, NO // 2), lambda f: (0, 0)),
            pl.BlockSpec((1, NO // 2), lambda f: (0, 0)),
        ],
        out_specs=pl.BlockSpec((1, VOCAB, NO // 2), lambda f: (f, 0, 0)),
        out_shape=jax.ShapeDtypeStruct((NF, VOCAB, NO // 2), jnp.uint32),
    )(emb_tables, w_e, w_o, b_e, b_o)
    return p.reshape(NF * VOCAB, NO // 2)


def _make_bag_kernel(bs):
    spw = bs // NW
    nchunk = spw // CH
    mesh = plsc.VectorSubcoreMesh(core_axis_name="c", subcore_axis_name="s")

    @functools.partial(
        pl.kernel,
        out_type=jax.ShapeDtypeStruct((bs, NO), jnp.float32),
        mesh=mesh,
        scratch_types=[
            pltpu.VMEM((nchunk, ROWSP), jnp.int32),
            pltpu.VMEM((ROWSP, NO // 2), jnp.uint32),
            pltpu.VMEM((ROWSP, NO // 2), jnp.uint32),
            pltpu.VMEM((spw, NO), jnp.float32),
            pltpu.SemaphoreType.DMA,
            pltpu.SemaphoreType.DMA,
        ],
    )
    def bag(idx_hbm, p_hbm, out_hbm, idx_v, rows0, rows1, out_v, sem0, sem1):
        cid = lax.axis_index("c")
        sid = lax.axis_index("s")
        w = sid * NC + cid

        pltpu.sync_copy(idx_hbm.at[w], idx_v)

        himask = jnp.full((LANES,), 0xFFFF0000, jnp.uint32)

        def compute_chunk(j, rows):
            def cc_body(cc, carry):
                col = pl.ds(cc * LANES, LANES)
                for s in range(CH):
                    x = rows[s * NF, col]
                    a0 = lax.bitcast_convert_type(x << 16, jnp.float32)
                    a1 = lax.bitcast_convert_type(x & himask, jnp.float32)
                    for f in range(1, NF):
                        x = rows[s * NF + f, col]
                        a0 = a0 + lax.bitcast_convert_type(x << 16, jnp.float32)
                        a1 = a1 + lax.bitcast_convert_type(x & himask, jnp.float32)
                    out_v[j * CH + s, pl.ds(cc * LANES, LANES)] = a0
                    out_v[j * CH + s, pl.ds(NO // 2 + cc * LANES, LANES)] = a1
                return carry

            lax.fori_loop(0, NO // (2 * LANES), cc_body, 0)

        pltpu.async_copy(p_hbm.at[idx_v.at[0]], rows0, sem0)

        def loop_body(jj, carry):
            j0 = 2 * jj
            j1 = j0 + 1
            pltpu.async_copy(p_hbm.at[idx_v.at[j1]], rows1, sem1)
            pltpu.make_async_copy(p_hbm.at[idx_v.at[j0]], rows0, sem0).wait()
            compute_chunk(j0, rows0)

            @pl.when(jj < nchunk // 2 - 1)
            def _():
                pltpu.async_copy(p_hbm.at[idx_v.at[j0 + 2]], rows0, sem0)

            pltpu.make_async_copy(p_hbm.at[idx_v.at[j1]], rows1, sem1).wait()
            compute_chunk(j1, rows1)
            return carry

        lax.fori_loop(0, nchunk // 2, loop_body, 0)

        pltpu.sync_copy(out_v, out_hbm.at[pl.ds(w * spw, spw)])

    return bag


def kernel(feature, edge_index, emb_tables, fc_W, fc_b):
    del edge_index
    bs = feature.shape[0]
    spw = bs // NW
    nchunk = spw // CH

    flat = feature.astype(jnp.int32) + (jnp.arange(NF, dtype=jnp.int32) * VOCAB)[None, :]
    idx = flat.reshape(NW, nchunk, ROWS)
    idx = jnp.pad(idx, ((0, 0), (0, 0), (0, ROWSP - ROWS)))

    p = _project_tables(emb_tables, fc_W, fc_b)
    return _make_bag_kernel(bs)(idx, p)
